# Initial kernel scaffold; baseline (speedup 1.0000x reference)
import jax
import jax.numpy as jnp
from jax.experimental import pallas as pl


def kernel(x, edge_attr, params, edge_index, batch):
    raise NotImplementedError("write your pallas kernel here")


# speedup vs baseline: 2.1411x; 2.1411x over previous
import functools

import jax
import jax.numpy as jnp
from jax import lax
from jax.experimental import pallas as pl
from jax.experimental.pallas import tpu as pltpu
from jax.experimental.pallas import tpu_sc as plsc

N = 10000
NP = 10240
E = 320000
ND = 128
ED = 16
H = 512
L = 4
G = 16

AUG = 24
HC = 128
NCH = H // HC
NSC = 2
NSUB = 16
CPS = NCH // NSC
EP = E // NSUB
B = 32
NB = EP // B
NROW = NP // NSUB

EB = 2000
NEB = E // EB
RB = 1024
NRB = NP // RB

BN_SCALE = 1.0 / (1.0 + 1e-5) ** 0.5
_PREC = lax.Precision.HIGHEST


def _dot(a, b):
    return jnp.dot(a, b, preferred_element_type=jnp.float32, precision=_PREC)


def _fold_body(ewaug_ref, lw_ref, lb_ref, out_ref):
    m = (lax.broadcasted_iota(jnp.int32, (AUG, 1), 0) == ED).astype(jnp.float32)
    for l in range(L):
        out_ref[l] = _dot(ewaug_ref[...], lw_ref[l]) + m * lb_ref[l][None, :]


def _fold_weights(ewaug, lw, lb):
    return pl.pallas_call(
        _fold_body,
        out_shape=jax.ShapeDtypeStruct((L, AUG, H), jnp.float32),
    )(ewaug, lw, lb)


def _edge_proj_body(ea_ref, waug_ref, *out_refs):
    a = ea_ref[...]
    for l in range(L):
        full = _dot(a, waug_ref[l])
        for c in range(NCH):
            out_refs[l][c] = full[:, c * HC:(c + 1) * HC]


def _edge_proj(ea_aug, waug):
    return pl.pallas_call(
        _edge_proj_body,
        grid=(NEB,),
        in_specs=[
            pl.BlockSpec((EB, AUG), lambda i: (i, 0)),
            pl.BlockSpec((L, AUG, H), lambda i: (0, 0, 0)),
        ],
        out_specs=[pl.BlockSpec((NCH, EB, HC), lambda i: (0, i, 0))] * L,
        out_shape=[jax.ShapeDtypeStruct((NCH, E, HC), jnp.float32)] * L,
    )(ea_aug, waug)


def _node_proj_body(x_ref, w_ref, b_ref, out_ref):
    full = _dot(x_ref[...], w_ref[...]) + b_ref[...]
    for c in range(NCH):
        out_ref[c] = full[:, c * HC:(c + 1) * HC]


def _node_proj(x, w, b):
    return pl.pallas_call(
        _node_proj_body,
        grid=(NRB,),
        in_specs=[
            pl.BlockSpec((RB, ND), lambda i: (i, 0)),
            pl.BlockSpec((ND, H), lambda i: (0, 0)),
            pl.BlockSpec((1, H), lambda i: (0, 0)),
        ],
        out_specs=pl.BlockSpec((NCH, RB, HC), lambda i: (0, i, 0)),
        out_shape=jax.ShapeDtypeStruct((NCH, NP, HC), jnp.float32),
    )(x, w, b)


def _sc_messages_body(h_hbm, ea_hbm, src_hbm, dst_hbm, z_hbm,
                      sring, dring, g0, g1, e0, e1, acc,
                      ss0, ss1, ds0, ds1, gs0, gs1, es0, es1):
    cid = lax.axis_index("c")
    sid = lax.axis_index("s")
    ebase = sid * EP
    ebase0 = sid * NB
    gbufs = (g0, g1)
    ebufs = (e0, e1)
    ssems = (ss0, ss1)
    dsems = (ds0, ds1)
    gsems = (gs0, gs1)
    esems = (es0, es1)

    for kk in range(CPS):
        c = cid * CPS + kk
        hoff = c * NP

        def idx_issue(q, r):
            pltpu.async_copy(src_hbm.at[pl.ds((ebase0 + q) * B, B)],
                             sring.at[r], ssems[r])
            pltpu.async_copy(dst_hbm.at[pl.ds((ebase0 + q) * B, B)],
                             dring.at[r], dsems[r])

        def gather(q, p, r):
            pltpu.make_async_copy(src_hbm.at[pl.ds(0, B)], sring.at[r],
                                  ssems[r]).wait()
            for u in range(B // 16):
                sl = pl.ds(u * 16, 16)
                sring[r, sl] = jnp.clip(sring[r, sl] + hoff, 0, NCH * NP - 1)
            pltpu.async_copy(h_hbm.at[sring.at[r]], gbufs[p], gsems[p])
            pltpu.async_copy(ea_hbm.at[pl.ds(c * E + ebase + q * B, B)],
                             ebufs[p], esems[p])

        def body(q, p, r):
            gb, eb = gbufs[p], ebufs[p]
            pltpu.make_async_copy(h_hbm.at[sring.at[r]], gb, gsems[p]).wait()
            pltpu.make_async_copy(ea_hbm.at[pl.ds(0, B)], eb, esems[p]).wait()

            @pl.loop(0, B)
            def _(rr):
                for u in range(HC // 16):
                    sl = pl.ds(u * 16, 16)
                    gb[rr, sl] = jnp.maximum(gb[rr, sl] + eb[rr, sl], 0.0)

            pltpu.make_async_copy(dst_hbm.at[pl.ds(0, B)], dring.at[r],
                                  dsems[r]).wait()
            for u in range(B // 16):
                sl = pl.ds(u * 16, 16)
                dring[r, sl] = jnp.clip(dring[r, sl], 0, NP - 1)
            pltpu.sync_copy(gb, acc.at[dring.at[r]], add=True)

        pltpu.sync_copy(h_hbm.at[pl.ds(hoff + sid * NROW, NROW)],
                        acc.at[pl.ds(sid * NROW, NROW)])
        plsc.subcore_barrier()

        idx_issue(0, 0)
        idx_issue(1, 1)
        gather(0, 0, 0)

        @pl.loop(0, NB - 1, step=2)
        def _(j):
            gather(j + 1, 1, 1)
            body(j, 0, 0)
            idx_issue(j + 2, 0)
            gather(j + 2, 0, 0)
            body(j + 1, 1, 1)

            @pl.when(j + 3 < NB)
            def _():
                idx_issue(j + 3, 1)

        body(NB - 1, 0, 0)

        plsc.subcore_barrier()
        pltpu.sync_copy(acc.at[pl.ds(sid * NROW, NROW)],
                        z_hbm.at[pl.ds(hoff + sid * NROW, NROW)])


@functools.cache
def _sc_messages_call():
    mesh = plsc.VectorSubcoreMesh(core_axis_name="c", subcore_axis_name="s",
                                  num_cores=NSC, num_subcores=NSUB)
    return pl.kernel(
        _sc_messages_body,
        out_type=jax.ShapeDtypeStruct((NCH * NP, HC), jnp.float32),
        mesh=mesh,
        scratch_types=[
            pltpu.VMEM((2, B), jnp.int32),
            pltpu.VMEM((2, B), jnp.int32),
            pltpu.VMEM((B, HC), jnp.float32),
            pltpu.VMEM((B, HC), jnp.float32),
            pltpu.VMEM((B, HC), jnp.float32),
            pltpu.VMEM((B, HC), jnp.float32),
            pltpu.VMEM_SHARED((NP, HC), jnp.float32),
        ] + [pltpu.SemaphoreType.DMA] * 8,
    )


def _mlp_body(z1_ref, h_ref, w1_ref, b1_ref, w2_ref, b2_ref, gb_ref, out_ref):
    z1 = jnp.concatenate([z1_ref[c] for c in range(NCH)], axis=1)
    hh = jnp.concatenate([h_ref[c] for c in range(NCH)], axis=1)
    u = jnp.maximum(_dot(z1, w1_ref[...]) + b1_ref[...], 0.0)
    v = _dot(u, w2_ref[...]) + b2_ref[...]
    z = jnp.maximum(v * (gb_ref[0:1, :] * BN_SCALE) + gb_ref[1:2, :], 0.0)
    hn = hh + z
    for c in range(NCH):
        out_ref[c] = hn[:, c * HC:(c + 1) * HC]


def _node_mlp(z1, h, w1, b1, w2, b2, gb):
    return pl.pallas_call(
        _mlp_body,
        grid=(NRB,),
        in_specs=[
            pl.BlockSpec((NCH, RB, HC), lambda i: (0, i, 0)),
            pl.BlockSpec((NCH, RB, HC), lambda i: (0, i, 0)),
            pl.BlockSpec((H, H), lambda i: (0, 0)),
            pl.BlockSpec((1, H), lambda i: (0, 0)),
            pl.BlockSpec((H, H), lambda i: (0, 0)),
            pl.BlockSpec((1, H), lambda i: (0, 0)),
            pl.BlockSpec((2, H), lambda i: (0, 0)),
        ],
        out_specs=pl.BlockSpec((NCH, RB, HC), lambda i: (0, i, 0)),
        out_shape=jax.ShapeDtypeStruct((NCH, NP, HC), jnp.float32),
    )(z1, h, w1, b1, w2, b2, gb)


def _pool_body(h_ref, b_ref, out_ref, sum_scr, cnt_scr, max_scr):
    i = pl.program_id(0)

    @pl.when(i == 0)
    def _():
        sum_scr[...] = jnp.zeros_like(sum_scr)
        cnt_scr[...] = jnp.zeros_like(cnt_scr)
        max_scr[...] = jnp.full_like(max_scr, -jnp.inf)

    hh = jnp.concatenate([h_ref[c] for c in range(NCH)], axis=1)
    bb = b_ref[0, 0, :]
    oh_t = (lax.broadcasted_iota(jnp.int32, (G, RB), 0)
            == bb[None, :]).astype(jnp.float32)
    sum_scr[...] += _dot(oh_t, hh)
    cnt_scr[...] += jnp.broadcast_to(jnp.sum(oh_t, axis=1)[:, None], (G, 128))
    for g in range(G):
        mg = jnp.where(bb[:, None] == g, hh, -jnp.inf)
        mx = jnp.max(mg, axis=0, keepdims=True)
        max_scr[g:g + 1, :] = jnp.maximum(max_scr[g:g + 1, :], mx)

    @pl.when(i == NRB - 1)
    def _():
        cnt = jnp.maximum(cnt_scr[:, 0:1], 1.0)
        out_ref[:, 0:H] = sum_scr[...] / cnt
        out_ref[:, H:2 * H] = max_scr[...]


def _pool(h, batch3):
    return pl.pallas_call(
        _pool_body,
        grid=(NRB,),
        in_specs=[
            pl.BlockSpec((NCH, RB, HC), lambda i: (0, i, 0)),
            pl.BlockSpec((1, 1, RB), lambda i: (i, 0, 0)),
        ],
        out_specs=pl.BlockSpec((G, 2 * H), lambda i: (0, 0)),
        out_shape=jax.ShapeDtypeStruct((G, 2 * H), jnp.float32),
        scratch_shapes=[
            pltpu.VMEM((G, H), jnp.float32),
            pltpu.VMEM((G, 128), jnp.float32),
            pltpu.VMEM((G, H), jnp.float32),
        ],
    )(h, batch3)


def kernel(x, edge_attr, params, edge_index, batch):
    ea_aug = jnp.concatenate(
        [edge_attr,
         jnp.ones((E, 1), jnp.float32),
         jnp.zeros((E, AUG - ED - 1), jnp.float32)], axis=1)
    ewaug = jnp.concatenate(
        [params['edge_W'],
         params['edge_b'][None, :],
         jnp.zeros((AUG - ED - 1, H), jnp.float32)], axis=0)
    lw = jnp.stack([params[f'lin_e{i}_W'] for i in range(L)])
    lb = jnp.stack([params[f'lin_e{i}_b'] for i in range(L)])

    waug = _fold_weights(ewaug, lw, lb)
    ea_list = _edge_proj(ea_aug, waug)
    x_pad = jnp.pad(x, ((0, NP - N), (0, 0)))
    h = _node_proj(x_pad, params['node_W'], params['node_b'][None, :])

    src1 = edge_index[0]
    dst1 = edge_index[1]

    for l in range(L):
        z2d = _sc_messages_call()(h.reshape(NCH * NP, HC),
                                  ea_list[l].reshape(NCH * E, HC), src1, dst1)
        z1 = z2d.reshape(NCH, NP, HC)
        h = _node_mlp(z1, h,
                      params[f'mlp{l}_W1'], params[f'mlp{l}_b1'][None, :],
                      params[f'mlp{l}_W2'], params[f'mlp{l}_b2'][None, :],
                      jnp.stack([params[f'bn{l}_gamma'],
                                 params[f'bn{l}_beta']]))

    batch_pad = jnp.pad(batch, (0, NP - N), constant_values=G)
    return _pool(h, batch_pad.reshape(NRB, 1, RB))


# speedup vs baseline: 2.5492x; 1.1906x over previous
import functools

import jax
import jax.numpy as jnp
from jax import lax
from jax.experimental import pallas as pl
from jax.experimental.pallas import tpu as pltpu
from jax.experimental.pallas import tpu_sc as plsc

N = 10000
NP = 10240
E = 320000
ND = 128
ED = 16
H = 512
L = 4
G = 16

AUG = 24
HC = 128
NCH = H // HC
NSC = 2
NSUB = 16
CPS = NCH // NSC
EP = E // NSUB
B = 32
NB = EP // B
NROW = NP // NSUB

EB = 2000
NEB = E // EB
RB = 1024
NRB = NP // RB

BN_SCALE = 1.0 / (1.0 + 1e-5) ** 0.5
_PREC = lax.Precision.HIGHEST


def _dot(a, b):
    return jnp.dot(a, b, preferred_element_type=jnp.float32, precision=_PREC)


def _fold_body(ewaug_ref, lw_ref, lb_ref, out_ref):
    m = (lax.broadcasted_iota(jnp.int32, (AUG, 1), 0) == ED).astype(jnp.float32)
    for l in range(L):
        out_ref[l] = _dot(ewaug_ref[...], lw_ref[l]) + m * lb_ref[l][None, :]


def _fold_weights(ewaug, lw, lb):
    return pl.pallas_call(
        _fold_body,
        out_shape=jax.ShapeDtypeStruct((L, AUG, H), jnp.float32),
    )(ewaug, lw, lb)


def _edge_proj_body(ea_ref, waug_ref, *out_refs):
    a = ea_ref[...]
    for l in range(L):
        full = _dot(a, waug_ref[l])
        for c in range(NCH):
            out_refs[l][c] = full[:, c * HC:(c + 1) * HC]


def _edge_proj(ea_aug, waug):
    return pl.pallas_call(
        _edge_proj_body,
        grid=(NEB,),
        in_specs=[
            pl.BlockSpec((EB, AUG), lambda i: (i, 0)),
            pl.BlockSpec((L, AUG, H), lambda i: (0, 0, 0)),
        ],
        out_specs=[pl.BlockSpec((NCH, EB, HC), lambda i: (0, i, 0))] * L,
        out_shape=[jax.ShapeDtypeStruct((NCH, E, HC), jnp.float32)] * L,
    )(ea_aug, waug)


def _node_proj_body(x_ref, w_ref, b_ref, out_ref):
    full = _dot(x_ref[...], w_ref[...]) + b_ref[...]
    for c in range(NCH):
        out_ref[c] = full[:, c * HC:(c + 1) * HC]


def _node_proj(x, w, b):
    return pl.pallas_call(
        _node_proj_body,
        grid=(NRB,),
        in_specs=[
            pl.BlockSpec((RB, ND), lambda i: (i, 0)),
            pl.BlockSpec((ND, H), lambda i: (0, 0)),
            pl.BlockSpec((1, H), lambda i: (0, 0)),
        ],
        out_specs=pl.BlockSpec((NCH, RB, HC), lambda i: (0, i, 0)),
        out_shape=jax.ShapeDtypeStruct((NCH, NP, HC), jnp.float32),
    )(x, w, b)


def _sc_messages_body(h_hbm, ea_hbm, src_hbm, dst_hbm, z_hbm,
                      sring, dring, didx, g0, g1, e0, e1, acc,
                      ss0, ss1, ss2, ss3, ds0, ds1, ds2, ds3,
                      gs0, gs1, es0, es1, sc0, sc1):
    cid = lax.axis_index("c")
    sid = lax.axis_index("s")
    ebase = sid * EP
    ebase0 = sid * NB
    gbufs = (g0, g1)
    ebufs = (e0, e1)
    ssems = (ss0, ss1, ss2, ss3)
    dsems = (ds0, ds1, ds2, ds3)
    gsems = (gs0, gs1)
    esems = (es0, es1)
    scsems = (sc0, sc1)

    for kk in range(CPS):
        c = cid * CPS + kk
        hoff = c * NP

        def idx_issue(q, r):
            pltpu.async_copy(src_hbm.at[pl.ds((ebase0 + q) * B, B)],
                             sring.at[r], ssems[r])
            pltpu.async_copy(dst_hbm.at[pl.ds((ebase0 + q) * B, B)],
                             dring.at[r], dsems[r])

        def gather(q, p, r, first=False):
            pltpu.make_async_copy(src_hbm.at[pl.ds(0, B)], sring.at[r],
                                  ssems[r]).wait()
            for u in range(B // 16):
                sl = pl.ds(u * 16, 16)
                sring[r, sl] = jnp.clip(sring[r, sl] + hoff, 0, NCH * NP - 1)
            if not first:
                @pl.when(q >= 2)
                def _():
                    pltpu.make_async_copy(gbufs[p], acc.at[didx.at[p]],
                                          scsems[p]).wait()
            pltpu.async_copy(h_hbm.at[sring.at[r]], gbufs[p], gsems[p])
            pltpu.async_copy(ea_hbm.at[pl.ds(c * E + ebase + q * B, B)],
                             ebufs[p], esems[p])

        def body(q, p, r):
            gb, eb = gbufs[p], ebufs[p]
            pltpu.make_async_copy(h_hbm.at[sring.at[r]], gb, gsems[p]).wait()
            pltpu.make_async_copy(ea_hbm.at[pl.ds(0, B)], eb, esems[p]).wait()

            @pl.loop(0, B)
            def _(rr):
                for u in range(HC // 16):
                    sl = pl.ds(u * 16, 16)
                    gb[rr, sl] = jnp.maximum(gb[rr, sl] + eb[rr, sl], 0.0)

            pltpu.make_async_copy(dst_hbm.at[pl.ds(0, B)], dring.at[r],
                                  dsems[r]).wait()
            for u in range(B // 16):
                sl = pl.ds(u * 16, 16)
                didx[p, sl] = jnp.clip(dring[r, sl], 0, NP - 1)
            pltpu.async_copy(gb, acc.at[didx.at[p]], scsems[p], add=True)

        pltpu.sync_copy(h_hbm.at[pl.ds(hoff + sid * NROW, NROW)],
                        acc.at[pl.ds(sid * NROW, NROW)])
        plsc.subcore_barrier()

        for r in range(4):
            idx_issue(r, r)
        gather(0, 0, 0, first=True)

        @pl.loop(0, NB - 1, step=4)
        def _(j):
            for t in range(4):
                q = j + t
                gather(q + 1, (t + 1) % 2, (t + 1) % 4)
                body(q, t % 2, t % 4)

                @pl.when(q + 4 < NB)
                def _():
                    idx_issue(q + 4, t % 4)

        body(NB - 1, 0, 0)
        pltpu.make_async_copy(gbufs[1], acc.at[didx.at[1]], scsems[1]).wait()
        pltpu.make_async_copy(gbufs[0], acc.at[didx.at[0]], scsems[0]).wait()

        plsc.subcore_barrier()
        pltpu.sync_copy(acc.at[pl.ds(sid * NROW, NROW)],
                        z_hbm.at[pl.ds(hoff + sid * NROW, NROW)])


@functools.cache
def _sc_messages_call():
    mesh = plsc.VectorSubcoreMesh(core_axis_name="c", subcore_axis_name="s",
                                  num_cores=NSC, num_subcores=NSUB)
    return pl.kernel(
        _sc_messages_body,
        out_type=jax.ShapeDtypeStruct((NCH * NP, HC), jnp.float32),
        mesh=mesh,
        scratch_types=[
            pltpu.VMEM((4, B), jnp.int32),
            pltpu.VMEM((4, B), jnp.int32),
            pltpu.VMEM((2, B), jnp.int32),
            pltpu.VMEM((B, HC), jnp.float32),
            pltpu.VMEM((B, HC), jnp.float32),
            pltpu.VMEM((B, HC), jnp.float32),
            pltpu.VMEM((B, HC), jnp.float32),
            pltpu.VMEM_SHARED((NP, HC), jnp.float32),
        ] + [pltpu.SemaphoreType.DMA] * 14,
    )


def _mlp_body(z1_ref, h_ref, w1_ref, b1_ref, w2_ref, b2_ref, gb_ref, out_ref):
    z1 = jnp.concatenate([z1_ref[c] for c in range(NCH)], axis=1)
    hh = jnp.concatenate([h_ref[c] for c in range(NCH)], axis=1)
    u = jnp.maximum(_dot(z1, w1_ref[...]) + b1_ref[...], 0.0)
    v = _dot(u, w2_ref[...]) + b2_ref[...]
    z = jnp.maximum(v * (gb_ref[0:1, :] * BN_SCALE) + gb_ref[1:2, :], 0.0)
    hn = hh + z
    for c in range(NCH):
        out_ref[c] = hn[:, c * HC:(c + 1) * HC]


def _node_mlp(z1, h, w1, b1, w2, b2, gb):
    return pl.pallas_call(
        _mlp_body,
        grid=(NRB,),
        in_specs=[
            pl.BlockSpec((NCH, RB, HC), lambda i: (0, i, 0)),
            pl.BlockSpec((NCH, RB, HC), lambda i: (0, i, 0)),
            pl.BlockSpec((H, H), lambda i: (0, 0)),
            pl.BlockSpec((1, H), lambda i: (0, 0)),
            pl.BlockSpec((H, H), lambda i: (0, 0)),
            pl.BlockSpec((1, H), lambda i: (0, 0)),
            pl.BlockSpec((2, H), lambda i: (0, 0)),
        ],
        out_specs=pl.BlockSpec((NCH, RB, HC), lambda i: (0, i, 0)),
        out_shape=jax.ShapeDtypeStruct((NCH, NP, HC), jnp.float32),
    )(z1, h, w1, b1, w2, b2, gb)


def _pool_body(h_ref, b_ref, out_ref, sum_scr, cnt_scr, max_scr):
    i = pl.program_id(0)

    @pl.when(i == 0)
    def _():
        sum_scr[...] = jnp.zeros_like(sum_scr)
        cnt_scr[...] = jnp.zeros_like(cnt_scr)
        max_scr[...] = jnp.full_like(max_scr, -jnp.inf)

    hh = jnp.concatenate([h_ref[c] for c in range(NCH)], axis=1)
    bb = b_ref[0, 0, :]
    oh_t = (lax.broadcasted_iota(jnp.int32, (G, RB), 0)
            == bb[None, :]).astype(jnp.float32)
    sum_scr[...] += _dot(oh_t, hh)
    cnt_scr[...] += jnp.broadcast_to(jnp.sum(oh_t, axis=1)[:, None], (G, 128))
    for g in range(G):
        mg = jnp.where(bb[:, None] == g, hh, -jnp.inf)
        mx = jnp.max(mg, axis=0, keepdims=True)
        max_scr[g:g + 1, :] = jnp.maximum(max_scr[g:g + 1, :], mx)

    @pl.when(i == NRB - 1)
    def _():
        cnt = jnp.maximum(cnt_scr[:, 0:1], 1.0)
        out_ref[:, 0:H] = sum_scr[...] / cnt
        out_ref[:, H:2 * H] = max_scr[...]


def _pool(h, batch3):
    return pl.pallas_call(
        _pool_body,
        grid=(NRB,),
        in_specs=[
            pl.BlockSpec((NCH, RB, HC), lambda i: (0, i, 0)),
            pl.BlockSpec((1, 1, RB), lambda i: (i, 0, 0)),
        ],
        out_specs=pl.BlockSpec((G, 2 * H), lambda i: (0, 0)),
        out_shape=jax.ShapeDtypeStruct((G, 2 * H), jnp.float32),
        scratch_shapes=[
            pltpu.VMEM((G, H), jnp.float32),
            pltpu.VMEM((G, 128), jnp.float32),
            pltpu.VMEM((G, H), jnp.float32),
        ],
    )(h, batch3)


def kernel(x, edge_attr, params, edge_index, batch):
    ea_aug = jnp.concatenate(
        [edge_attr,
         jnp.ones((E, 1), jnp.float32),
         jnp.zeros((E, AUG - ED - 1), jnp.float32)], axis=1)
    ewaug = jnp.concatenate(
        [params['edge_W'],
         params['edge_b'][None, :],
         jnp.zeros((AUG - ED - 1, H), jnp.float32)], axis=0)
    lw = jnp.stack([params[f'lin_e{i}_W'] for i in range(L)])
    lb = jnp.stack([params[f'lin_e{i}_b'] for i in range(L)])

    waug = _fold_weights(ewaug, lw, lb)
    ea_list = _edge_proj(ea_aug, waug)
    x_pad = jnp.pad(x, ((0, NP - N), (0, 0)))
    h = _node_proj(x_pad, params['node_W'], params['node_b'][None, :])

    src1 = edge_index[0]
    dst1 = edge_index[1]

    for l in range(L):
        z2d = _sc_messages_call()(h.reshape(NCH * NP, HC),
                                  ea_list[l].reshape(NCH * E, HC), src1, dst1)
        z1 = z2d.reshape(NCH, NP, HC)
        h = _node_mlp(z1, h,
                      params[f'mlp{l}_W1'], params[f'mlp{l}_b1'][None, :],
                      params[f'mlp{l}_W2'], params[f'mlp{l}_b2'][None, :],
                      jnp.stack([params[f'bn{l}_gamma'],
                                 params[f'bn{l}_beta']]))

    batch_pad = jnp.pad(batch, (0, NP - N), constant_values=G)
    return _pool(h, batch_pad.reshape(NRB, 1, RB))
